# Initial kernel scaffold; baseline (speedup 1.0000x reference)
import jax
import jax.numpy as jnp
from jax.experimental import pallas as pl


def kernel(vocab_embeddings, random_ids, weight_logits):
    raise NotImplementedError("write your pallas kernel here")


# speedup vs baseline: 2.2396x; 2.2396x over previous
import functools

import jax
import jax.numpy as jnp
from jax import lax
from jax.experimental import pallas as pl
from jax.experimental.pallas import tpu as pltpu
from jax.experimental.pallas import tpu_sc as plsc

NC = 2
NS = 16
L = 16
NW = NC * NS

K = 3
D = 1024
CH = 8


def _body(table_hbm, ids_hbm, logits_hbm, out_hbm,
          idx_v, logits_v, weights_v, rows_v, out_v, sem):
    ntok = out_hbm.shape[0]
    tpw = ntok // NW
    nch = tpw // CH

    cid = lax.axis_index("c")
    sid = lax.axis_index("s")
    wid = sid * NC + cid
    tok0 = wid * tpw

    pltpu.sync_copy(ids_hbm.at[pl.ds(wid * nch, nch)], idx_v)
    pltpu.sync_copy(logits_hbm.at[pl.ds(wid * tpw * K, tpw * K)], logits_v)

    for tg in range(tpw // L):
        tvec = jnp.arange(L, dtype=jnp.int32) * K + (tg * L * K)
        w0 = plsc.load_gather(logits_v, [tvec])
        w1 = plsc.load_gather(logits_v, [tvec + 1])
        w2 = plsc.load_gather(logits_v, [tvec + 2])
        m = jnp.maximum(w0, jnp.maximum(w1, w2))
        e0 = jnp.exp(w0 - m)
        e1 = jnp.exp(w1 - m)
        e2 = jnp.exp(w2 - m)
        inv = 1.0 / (e0 + e1 + e2)
        weights_v[pl.ds(0 * tpw + tg * L, L)] = e0 * inv
        weights_v[pl.ds(1 * tpw + tg * L, L)] = e1 * inv
        weights_v[pl.ds(2 * tpw + tg * L, L)] = e2 * inv

    def chunk_body(c, carry):
        pltpu.async_copy(table_hbm.at[idx_v.at[c]], rows_v, sem).wait()
        for t in range(CH):
            tix = c * CH + t
            w0 = plsc.load_gather(
                weights_v, [jnp.full((L,), 0 * tpw + tix, jnp.int32)])
            w1 = plsc.load_gather(
                weights_v, [jnp.full((L,), 1 * tpw + tix, jnp.int32)])
            w2 = plsc.load_gather(
                weights_v, [jnp.full((L,), 2 * tpw + tix, jnp.int32)])

            def d_body(d, carry2, t=t, w0=w0, w1=w1, w2=w2):
                sl = pl.ds(d * L, L)
                r0 = rows_v[K * t + 0, sl]
                r1 = rows_v[K * t + 1, sl]
                r2 = rows_v[K * t + 2, sl]
                out_v[t, sl] = w0 * r0 + w1 * r1 + w2 * r2
                return carry2

            lax.fori_loop(0, D // L, d_body, 0)
        pltpu.sync_copy(out_v, out_hbm.at[pl.ds(tok0 + c * CH, CH)])
        return carry

    lax.fori_loop(0, nch, chunk_body, 0)


def kernel(vocab_embeddings, random_ids, weight_logits):
    B, S, k = random_ids.shape
    ntok = B * S
    assert k == K and vocab_embeddings.shape[1] == D
    assert ntok % (NW * CH) == 0
    tpw = ntok // NW
    nch = tpw // CH

    ids2d = random_ids.reshape(ntok // CH, CH * K)
    logits_flat = weight_logits.reshape(ntok * K)

    mesh = plsc.VectorSubcoreMesh(
        core_axis_name="c", subcore_axis_name="s",
        num_cores=NC, num_subcores=NS)

    run = pl.kernel(
        _body,
        out_type=jax.ShapeDtypeStruct((ntok, D), jnp.float32),
        mesh=mesh,
        scratch_types=[
            pltpu.VMEM((nch, CH * K), jnp.int32),
            pltpu.VMEM((tpw * K,), jnp.float32),
            pltpu.VMEM((K * tpw,), jnp.float32),
            pltpu.VMEM((CH * K, D), jnp.float32),
            pltpu.VMEM((CH, D), jnp.float32),
            pltpu.SemaphoreType.DMA,
        ],
        compiler_params=pltpu.CompilerParams(needs_layout_passes=False),
    )
    out = run(vocab_embeddings, ids2d, logits_flat)
    return out.reshape(B, S, D)


# speedup vs baseline: 3.5013x; 1.5634x over previous
import functools

import jax
import jax.numpy as jnp
from jax import lax
from jax.experimental import pallas as pl
from jax.experimental.pallas import tpu as pltpu
from jax.experimental.pallas import tpu_sc as plsc

NC = 2
NS = 16
L = 16
NW = NC * NS

K = 3
D = 1024
CH = 8


UNROLL = 16


def _body(table_hbm, ids_hbm, logits_hbm, out_hbm,
          idx_v, logits_v, weights_v, rows_v, out_v,
          gsem0, gsem1, osem0, osem1):
    ntok = out_hbm.shape[0]
    tpw = ntok // NW
    nch = tpw // CH
    gsem = (gsem0, gsem1)
    osem = (osem0, osem1)

    cid = lax.axis_index("c")
    sid = lax.axis_index("s")
    wid = sid * NC + cid
    tok0 = wid * tpw

    pltpu.sync_copy(ids_hbm.at[pl.ds(wid * nch, nch)], idx_v)
    pltpu.sync_copy(logits_hbm.at[pl.ds(wid * tpw * K, tpw * K)], logits_v)

    for tg in range(tpw // L):
        tvec = jnp.arange(L, dtype=jnp.int32) * K + (tg * L * K)
        w0 = plsc.load_gather(logits_v, [tvec])
        w1 = plsc.load_gather(logits_v, [tvec + 1])
        w2 = plsc.load_gather(logits_v, [tvec + 2])
        m = jnp.maximum(w0, jnp.maximum(w1, w2))
        e0 = jnp.exp(w0 - m)
        e1 = jnp.exp(w1 - m)
        e2 = jnp.exp(w2 - m)
        inv = 1.0 / (e0 + e1 + e2)
        weights_v[pl.ds(0 * tpw + tg * L, L)] = e0 * inv
        weights_v[pl.ds(1 * tpw + tg * L, L)] = e1 * inv
        weights_v[pl.ds(2 * tpw + tg * L, L)] = e2 * inv

    pltpu.async_copy(table_hbm.at[idx_v.at[0]], rows_v.at[0], gsem[0])

    def pair_body(c2, carry):
        for b in range(2):
            c = 2 * c2 + b
            nxt = jnp.minimum(c + 1, nch - 1)
            pltpu.async_copy(
                table_hbm.at[idx_v.at[nxt]], rows_v.at[1 - b], gsem[1 - b])
            pltpu.make_async_copy(
                table_hbm.at[idx_v.at[c]], rows_v.at[b], gsem[b]).wait()

            ws = []
            for t in range(CH):
                tix = c * CH + t
                ws.append([
                    plsc.load_gather(
                        weights_v, [jnp.full((L,), kk * tpw + tix, jnp.int32)])
                    for kk in range(K)])

            @pl.when(c2 > 0)
            def _():
                pltpu.make_async_copy(
                    out_v.at[b], out_hbm.at[pl.ds(tok0, CH)], osem[b]).wait()

            def g_body(g, carry2):
                for u in range(UNROLL):
                    sl = pl.ds((g * UNROLL + u) * L, L)
                    for t in range(CH):
                        r0 = rows_v[b, K * t + 0, sl]
                        r1 = rows_v[b, K * t + 1, sl]
                        r2 = rows_v[b, K * t + 2, sl]
                        out_v[b, t, sl] = (
                            ws[t][0] * r0 + ws[t][1] * r1 + ws[t][2] * r2)
                return carry2

            lax.fori_loop(0, (D // L) // UNROLL, g_body, 0)
            pltpu.async_copy(
                out_v.at[b], out_hbm.at[pl.ds(tok0 + c * CH, CH)], osem[b])
        return carry

    lax.fori_loop(0, nch // 2, pair_body, 0)

    pltpu.make_async_copy(
        table_hbm.at[idx_v.at[0]], rows_v.at[0], gsem[0]).wait()
    pltpu.make_async_copy(
        out_v.at[0], out_hbm.at[pl.ds(tok0, CH)], osem[0]).wait()
    pltpu.make_async_copy(
        out_v.at[1], out_hbm.at[pl.ds(tok0, CH)], osem[1]).wait()


def kernel(vocab_embeddings, random_ids, weight_logits):
    B, S, k = random_ids.shape
    ntok = B * S
    assert k == K and vocab_embeddings.shape[1] == D
    assert ntok % (NW * CH) == 0
    tpw = ntok // NW
    nch = tpw // CH

    ids2d = random_ids.reshape(ntok // CH, CH * K)
    logits_flat = weight_logits.reshape(ntok * K)

    mesh = plsc.VectorSubcoreMesh(
        core_axis_name="c", subcore_axis_name="s",
        num_cores=NC, num_subcores=NS)

    run = pl.kernel(
        _body,
        out_type=jax.ShapeDtypeStruct((ntok, D), jnp.float32),
        mesh=mesh,
        scratch_types=[
            pltpu.VMEM((nch, CH * K), jnp.int32),
            pltpu.VMEM((tpw * K,), jnp.float32),
            pltpu.VMEM((K * tpw,), jnp.float32),
            pltpu.VMEM((2, CH * K, D), jnp.float32),
            pltpu.VMEM((2, CH, D), jnp.float32),
            pltpu.SemaphoreType.DMA,
            pltpu.SemaphoreType.DMA,
            pltpu.SemaphoreType.DMA,
            pltpu.SemaphoreType.DMA,
        ],
        compiler_params=pltpu.CompilerParams(needs_layout_passes=False),
    )
    out = run(vocab_embeddings, ids2d, logits_flat)
    return out.reshape(B, S, D)


# speedup vs baseline: 3.5516x; 1.0144x over previous
import functools

import jax
import jax.numpy as jnp
from jax import lax
from jax.experimental import pallas as pl
from jax.experimental.pallas import tpu as pltpu
from jax.experimental.pallas import tpu_sc as plsc

NC = 2
NS = 16
L = 16
NW = NC * NS

K = 3
D = 1024
CH = 8


UNROLL = 16


def _body(table_hbm, ids_hbm, logits_hbm, out_hbm,
          idx_v, logits_v, weights_v, rows_v, out_v,
          gsem0, gsem1, osem0, osem1):
    ntok = out_hbm.shape[0]
    tpw = ntok // NW
    nch = tpw // CH
    gsem = (gsem0, gsem1)
    osem = (osem0, osem1)

    cid = lax.axis_index("c")
    sid = lax.axis_index("s")
    wid = sid * NC + cid
    tok0 = wid * tpw

    pltpu.sync_copy(ids_hbm.at[pl.ds(wid * nch, nch)], idx_v)
    pltpu.sync_copy(logits_hbm.at[pl.ds(wid * tpw * K, tpw * K)], logits_v)

    for tg in range(tpw // L):
        tvec = jnp.arange(L, dtype=jnp.int32) * K + (tg * L * K)
        w0 = plsc.load_gather(logits_v, [tvec])
        w1 = plsc.load_gather(logits_v, [tvec + 1])
        w2 = plsc.load_gather(logits_v, [tvec + 2])
        m = jnp.maximum(w0, jnp.maximum(w1, w2))
        e0 = jnp.exp(w0 - m)
        e1 = jnp.exp(w1 - m)
        e2 = jnp.exp(w2 - m)
        inv = 1.0 / (e0 + e1 + e2)
        weights_v[pl.ds(0 * tpw + tg * L, L)] = e0 * inv
        weights_v[pl.ds(1 * tpw + tg * L, L)] = e1 * inv
        weights_v[pl.ds(2 * tpw + tg * L, L)] = e2 * inv

    pltpu.async_copy(table_hbm.at[idx_v.at[0]], rows_v.at[0], gsem[0])

    def pair_body(c2, carry):
        for b in range(2):
            c = 2 * c2 + b
            nxt = jnp.minimum(c + 1, nch - 1)
            pltpu.async_copy(
                table_hbm.at[idx_v.at[nxt]], rows_v.at[1 - b], gsem[1 - b])
            pltpu.make_async_copy(
                table_hbm.at[idx_v.at[c]], rows_v.at[b], gsem[b]).wait()

            @pl.when(c2 > 0)
            def _():
                pltpu.make_async_copy(
                    out_v.at[b], out_hbm.at[pl.ds(tok0, CH)], osem[b]).wait()

            for t in range(CH):
                tix = c * CH + t
                w0, w1, w2 = (
                    plsc.load_gather(
                        weights_v, [jnp.full((L,), kk * tpw + tix, jnp.int32)])
                    for kk in range(K))

                def g_body(g, carry2, t=t, w0=w0, w1=w1, w2=w2):
                    for u in range(UNROLL):
                        sl = pl.ds((g * UNROLL + u) * L, L)
                        r0 = rows_v[b, K * t + 0, sl]
                        r1 = rows_v[b, K * t + 1, sl]
                        r2 = rows_v[b, K * t + 2, sl]
                        out_v[b, t, sl] = w0 * r0 + w1 * r1 + w2 * r2
                    return carry2

                lax.fori_loop(0, (D // L) // UNROLL, g_body, 0)
            pltpu.async_copy(
                out_v.at[b], out_hbm.at[pl.ds(tok0 + c * CH, CH)], osem[b])
        return carry

    lax.fori_loop(0, nch // 2, pair_body, 0)

    pltpu.make_async_copy(
        table_hbm.at[idx_v.at[0]], rows_v.at[0], gsem[0]).wait()
    pltpu.make_async_copy(
        out_v.at[0], out_hbm.at[pl.ds(tok0, CH)], osem[0]).wait()
    pltpu.make_async_copy(
        out_v.at[1], out_hbm.at[pl.ds(tok0, CH)], osem[1]).wait()


def kernel(vocab_embeddings, random_ids, weight_logits):
    B, S, k = random_ids.shape
    ntok = B * S
    assert k == K and vocab_embeddings.shape[1] == D
    assert ntok % (NW * CH) == 0
    tpw = ntok // NW
    nch = tpw // CH

    ids2d = random_ids.reshape(ntok // CH, CH * K)
    logits_flat = weight_logits.reshape(ntok * K)

    mesh = plsc.VectorSubcoreMesh(
        core_axis_name="c", subcore_axis_name="s",
        num_cores=NC, num_subcores=NS)

    run = pl.kernel(
        _body,
        out_type=jax.ShapeDtypeStruct((ntok, D), jnp.float32),
        mesh=mesh,
        scratch_types=[
            pltpu.VMEM((nch, CH * K), jnp.int32),
            pltpu.VMEM((tpw * K,), jnp.float32),
            pltpu.VMEM((K * tpw,), jnp.float32),
            pltpu.VMEM((2, CH * K, D), jnp.float32),
            pltpu.VMEM((2, CH, D), jnp.float32),
            pltpu.SemaphoreType.DMA,
            pltpu.SemaphoreType.DMA,
            pltpu.SemaphoreType.DMA,
            pltpu.SemaphoreType.DMA,
        ],
        compiler_params=pltpu.CompilerParams(needs_layout_passes=False),
    )
    out = run(vocab_embeddings, ids2d, logits_flat)
    return out.reshape(B, S, D)


# speedup vs baseline: 4.0890x; 1.1513x over previous
import functools

import jax
import jax.numpy as jnp
from jax import lax
from jax.experimental import pallas as pl
from jax.experimental.pallas import tpu as pltpu
from jax.experimental.pallas import tpu_sc as plsc

NC = 2
NS = 16
L = 16
NW = NC * NS

K = 3
D = 1024
CH = 8


UNROLL = 16


def _body(table_hbm, ids_hbm, logits_hbm, out_hbm,
          idx_v, logits_v, weights_v, rows_v, out_v,
          gsem0, gsem1, osem0, osem1):
    ntok = out_hbm.shape[0]
    tpw = ntok // NW
    nch = tpw // CH
    gsem = (gsem0, gsem1)
    osem = (osem0, osem1)

    cid = lax.axis_index("c")
    sid = lax.axis_index("s")
    wid = sid * NC + cid
    tok0 = wid * tpw

    pltpu.sync_copy(ids_hbm.at[pl.ds(wid * nch, nch)], idx_v)
    pltpu.sync_copy(logits_hbm.at[pl.ds(wid * tpw * K, tpw * K)], logits_v)

    for tg in range(tpw // L):
        tvec = jnp.arange(L, dtype=jnp.int32) * K + (tg * L * K)
        w0 = plsc.load_gather(logits_v, [tvec])
        w1 = plsc.load_gather(logits_v, [tvec + 1])
        w2 = plsc.load_gather(logits_v, [tvec + 2])
        m = jnp.maximum(w0, jnp.maximum(w1, w2))
        e0 = jnp.exp(w0 - m)
        e1 = jnp.exp(w1 - m)
        e2 = jnp.exp(w2 - m)
        inv = 1.0 / (e0 + e1 + e2)
        weights_v[pl.ds(0 * tpw + tg * L, L)] = e0 * inv
        weights_v[pl.ds(1 * tpw + tg * L, L)] = e1 * inv
        weights_v[pl.ds(2 * tpw + tg * L, L)] = e2 * inv

    pltpu.async_copy(table_hbm.at[idx_v.at[0]], rows_v.at[0], gsem[0])

    def pair_body(c2, carry):
        for b in range(2):
            c = 2 * c2 + b
            nxt = jnp.minimum(c + 1, nch - 1)
            pltpu.async_copy(
                table_hbm.at[idx_v.at[nxt]], rows_v.at[1 - b], gsem[1 - b])
            pltpu.make_async_copy(
                table_hbm.at[idx_v.at[c]], rows_v.at[b], gsem[b]).wait()

            @pl.when(c2 > 0)
            def _():
                pltpu.make_async_copy(
                    out_v.at[b], out_hbm.at[pl.ds(tok0, CH)], osem[b]).wait()

            for t in range(CH):
                tix = c * CH + t
                w0, w1, w2 = (
                    plsc.load_gather(
                        weights_v, [jnp.full((L,), kk * tpw + tix, jnp.int32)])
                    for kk in range(K))

                def d_body(d, t=t, w0=w0, w1=w1, w2=w2):
                    sl = pl.ds(d * L, L)
                    r0 = rows_v[b, K * t + 0, sl]
                    r1 = rows_v[b, K * t + 1, sl]
                    r2 = rows_v[b, K * t + 2, sl]
                    out_v[b, t, sl] = w0 * r0 + w1 * r1 + w2 * r2

                plsc.parallel_loop(0, D // L, unroll=UNROLL)(d_body)
            pltpu.async_copy(
                out_v.at[b], out_hbm.at[pl.ds(tok0 + c * CH, CH)], osem[b])
        return carry

    lax.fori_loop(0, nch // 2, pair_body, 0)

    pltpu.make_async_copy(
        table_hbm.at[idx_v.at[0]], rows_v.at[0], gsem[0]).wait()
    pltpu.make_async_copy(
        out_v.at[0], out_hbm.at[pl.ds(tok0, CH)], osem[0]).wait()
    pltpu.make_async_copy(
        out_v.at[1], out_hbm.at[pl.ds(tok0, CH)], osem[1]).wait()


def kernel(vocab_embeddings, random_ids, weight_logits):
    B, S, k = random_ids.shape
    ntok = B * S
    assert k == K and vocab_embeddings.shape[1] == D
    assert ntok % (NW * CH) == 0
    tpw = ntok // NW
    nch = tpw // CH

    ids2d = random_ids.reshape(ntok // CH, CH * K)
    logits_flat = weight_logits.reshape(ntok * K)

    mesh = plsc.VectorSubcoreMesh(
        core_axis_name="c", subcore_axis_name="s",
        num_cores=NC, num_subcores=NS)

    run = pl.kernel(
        _body,
        out_type=jax.ShapeDtypeStruct((ntok, D), jnp.float32),
        mesh=mesh,
        scratch_types=[
            pltpu.VMEM((nch, CH * K), jnp.int32),
            pltpu.VMEM((tpw * K,), jnp.float32),
            pltpu.VMEM((K * tpw,), jnp.float32),
            pltpu.VMEM((2, CH * K, D), jnp.float32),
            pltpu.VMEM((2, CH, D), jnp.float32),
            pltpu.SemaphoreType.DMA,
            pltpu.SemaphoreType.DMA,
            pltpu.SemaphoreType.DMA,
            pltpu.SemaphoreType.DMA,
        ],
        compiler_params=pltpu.CompilerParams(needs_layout_passes=False),
    )
    out = run(vocab_embeddings, ids2d, logits_flat)
    return out.reshape(B, S, D)
